# baseline (speedup 1.0000x reference)
import functools

import jax
import jax.numpy as jnp
from jax import lax
from jax.experimental import pallas as pl
from jax.experimental.pallas import tpu as pltpu
from jax.experimental.pallas import tpu_sc as plsc

B = 4096
L = 200
D = 64
N = B * L
NW = 32
CH = 100
ROWS_PER_W = N // NW
NCH = ROWS_PER_W // CH
LANES = 16
VECS_PER_ROW = D // LANES


def _body(x_hbm, tok_hbm, pos_hbm, out_hbm, idx_v, rows_v, pos_v, sem):
    wid = lax.axis_index("s") * 2 + lax.axis_index("c")
    pltpu.sync_copy(x_hbm.at[pl.ds(wid * NCH, NCH)], idx_v)
    pltpu.sync_copy(pos_hbm, pos_v)

    def chunk(c, carry):
        gchunk = wid * NCH + c
        pltpu.async_copy(tok_hbm.at[idx_v.at[c]], rows_v, sem).wait()
        pbase = (c % 2) * (CH * D)

        def add_row(r, carry2):
            for d in range(VECS_PER_ROW):
                p = pos_v[pl.ds(pbase + r * D + d * LANES, LANES)]
                plsc.addupdate(rows_v.at[r, pl.ds(d * LANES, LANES)], p)
            return carry2

        lax.fori_loop(0, CH, add_row, 0, unroll=2)
        pltpu.sync_copy(rows_v, out_hbm.at[gchunk])
        return carry

    lax.fori_loop(0, NCH, chunk, 0)


@jax.jit
def _embed(x2d, token_table, pos_flat):
    mesh = plsc.VectorSubcoreMesh(core_axis_name="c", subcore_axis_name="s")
    kfn = pl.kernel(
        _body,
        out_type=jax.ShapeDtypeStruct((N // CH, CH, D), jnp.float32),
        mesh=mesh,
        scratch_types=[
            pltpu.VMEM((NCH, CH), jnp.int32),
            pltpu.VMEM((CH, D), jnp.float32),
            pltpu.VMEM((L * D,), jnp.float32),
            pltpu.SemaphoreType.DMA,
        ],
        compiler_params=pltpu.CompilerParams(use_tc_tiling_on_sc=False),
    )
    return kfn(x2d, token_table, pos_flat)


def kernel(x, token_table, pos_table):
    x2d = x.reshape(N // CH, CH).astype(jnp.int32)
    pos_flat = pos_table.reshape(L * D)
    out = _embed(x2d, token_table, pos_flat)
    return out.reshape(B, L, D)


# speedup vs baseline: 1.4799x; 1.4799x over previous
import jax
import jax.numpy as jnp
from jax import lax
from jax.experimental import pallas as pl
from jax.experimental.pallas import tpu as pltpu
from jax.experimental.pallas import tpu_sc as plsc

B = 4096
L = 200
D = 64
N = B * L
NW = 32
CH = 100
ROWS_PER_W = N // NW
NCH = ROWS_PER_W // CH
K = 4
NG = NCH // K
NGH = NG // 2
LANES = 16
VECS_PER_ROW = D // LANES


def _body(x_hbm, tok_hbm, pos_hbm, out_hbm, idx_v, rows_v, pos_v,
          sg0, sg1, sw0, sw1):
    wid = lax.axis_index("s") * 2 + lax.axis_index("c")
    semg = (sg0, sg1)
    semw = (sw0, sw1)
    pltpu.sync_copy(x_hbm.at[pl.ds(wid * NCH, NCH)], idx_v)
    pltpu.sync_copy(pos_hbm, pos_v)

    def fire_gathers(g, p):
        for b in range(K):
            c = g * K + b
            pltpu.async_copy(tok_hbm.at[idx_v.at[c]], rows_v.at[p, b], semg[p])

    def drain_gathers(p):
        for b in range(K):
            pltpu.make_async_copy(
                tok_hbm.at[pl.ds(0, CH)], rows_v.at[p, b], semg[p]).wait()

    def add_pos(p, b):
        pbase = (b % 2) * (CH * D)

        def row(r, carry):
            for d in range(VECS_PER_ROW):
                pv = pos_v[pl.ds(pbase + r * D + d * LANES, LANES)]
                plsc.addupdate(rows_v.at[p, b, r, pl.ds(d * LANES, LANES)], pv)
            return carry

        lax.fori_loop(0, CH, row, 0, unroll=4)

    def process_group(g, p):
        base = wid * NCH + g * K
        for b in range(K):
            add_pos(p, b)
            pltpu.async_copy(rows_v.at[p, b], out_hbm.at[base + b], semw[p])

    def drain_writes(p):
        for b in range(K):
            pltpu.make_async_copy(
                rows_v.at[p, b], out_hbm.at[0], semw[p]).wait()

    fire_gathers(0, 0)

    def step(t, carry):
        @pl.when(t > 0)
        def _():
            drain_writes(1)

        fire_gathers(2 * t + 1, 1)
        drain_gathers(0)
        process_group(2 * t, 0)

        @pl.when(t < NGH - 1)
        def _():
            drain_writes(0)
            fire_gathers(2 * t + 2, 0)

        drain_gathers(1)
        process_group(2 * t + 1, 1)
        return carry

    lax.fori_loop(0, NGH, step, 0)
    drain_writes(0)
    drain_writes(1)


@jax.jit
def _embed(x2d, token_table, pos_flat):
    mesh = plsc.VectorSubcoreMesh(core_axis_name="c", subcore_axis_name="s")
    kfn = pl.kernel(
        _body,
        out_type=jax.ShapeDtypeStruct((N // CH, CH, D), jnp.float32),
        mesh=mesh,
        scratch_types=[
            pltpu.VMEM((NCH, CH), jnp.int32),
            pltpu.VMEM((2, K, CH, D), jnp.float32),
            pltpu.VMEM((L * D,), jnp.float32),
            pltpu.SemaphoreType.DMA,
            pltpu.SemaphoreType.DMA,
            pltpu.SemaphoreType.DMA,
            pltpu.SemaphoreType.DMA,
        ],
        compiler_params=pltpu.CompilerParams(use_tc_tiling_on_sc=False),
    )
    return kfn(x2d, token_table, pos_flat)


def kernel(x, token_table, pos_table):
    x2d = x.reshape(N // CH, CH).astype(jnp.int32)
    pos_flat = pos_table.reshape(L * D)
    out = _embed(x2d, token_table, pos_flat)
    return out.reshape(B, L, D)


# speedup vs baseline: 1.4893x; 1.0064x over previous
import jax
import jax.numpy as jnp
from jax import lax
from jax.experimental import pallas as pl
from jax.experimental.pallas import tpu as pltpu
from jax.experimental.pallas import tpu_sc as plsc

B = 4096
L = 200
D = 64
N = B * L
NW = 32
CH = 100
ROWS_PER_W = N // NW
NCH = ROWS_PER_W // CH
K = 4
NG = NCH // K
NGH = NG // 2
LANES = 16
VECS_PER_ROW = D // LANES


def _body(x_hbm, tok_hbm, pos_hbm, out_hbm, idx_v, rows_v, pos_v,
          sg0, sg1, sw0, sw1):
    wid = lax.axis_index("s") * 2 + lax.axis_index("c")
    semg = (sg0, sg1)
    semw = (sw0, sw1)
    pltpu.sync_copy(x_hbm.at[pl.ds(wid * NCH, NCH)], idx_v)
    pltpu.sync_copy(pos_hbm, pos_v)

    def fire_gathers(g, p):
        for b in range(K):
            c = g * K + b
            pltpu.async_copy(tok_hbm.at[idx_v.at[c]], rows_v.at[p, b], semg[p])

    def drain_gathers(p):
        for b in range(K):
            pltpu.make_async_copy(
                tok_hbm.at[pl.ds(0, CH)], rows_v.at[p, b], semg[p]).wait()

    def add_pos(p, b):
        pbase = (b % 2) * (CH * D)

        def row(r, carry):
            for d in range(VECS_PER_ROW):
                pv = pos_v[pl.ds(pbase + r * D + d * LANES, LANES)]
                plsc.addupdate(rows_v.at[p, b, r, pl.ds(d * LANES, LANES)], pv)
            return carry

        lax.fori_loop(0, CH, row, 0, unroll=4)

    def process_group(g, p):
        base = wid * NCH + g * K
        for b in range(K):
            pltpu.async_copy(rows_v.at[p, b], out_hbm.at[base + b], semw[p])

    def drain_writes(p):
        for b in range(K):
            pltpu.make_async_copy(
                rows_v.at[p, b], out_hbm.at[0], semw[p]).wait()

    fire_gathers(0, 0)

    def step(t, carry):
        @pl.when(t > 0)
        def _():
            drain_writes(1)

        fire_gathers(2 * t + 1, 1)
        drain_gathers(0)
        process_group(2 * t, 0)

        @pl.when(t < NGH - 1)
        def _():
            drain_writes(0)
            fire_gathers(2 * t + 2, 0)

        drain_gathers(1)
        process_group(2 * t + 1, 1)
        return carry

    lax.fori_loop(0, NGH, step, 0)
    drain_writes(0)
    drain_writes(1)


@jax.jit
def _embed(x2d, token_table, pos_flat):
    mesh = plsc.VectorSubcoreMesh(core_axis_name="c", subcore_axis_name="s")
    kfn = pl.kernel(
        _body,
        out_type=jax.ShapeDtypeStruct((N // CH, CH, D), jnp.float32),
        mesh=mesh,
        scratch_types=[
            pltpu.VMEM((NCH, CH), jnp.int32),
            pltpu.VMEM((2, K, CH, D), jnp.float32),
            pltpu.VMEM((L * D,), jnp.float32),
            pltpu.SemaphoreType.DMA,
            pltpu.SemaphoreType.DMA,
            pltpu.SemaphoreType.DMA,
            pltpu.SemaphoreType.DMA,
        ],
        compiler_params=pltpu.CompilerParams(use_tc_tiling_on_sc=False),
    )
    return kfn(x2d, token_table, pos_flat)


def kernel(x, token_table, pos_table):
    x2d = x.reshape(N // CH, CH).astype(jnp.int32)
    pos_flat = pos_table.reshape(L * D)
    out = _embed(x2d, token_table, pos_flat)
    return out.reshape(B, L, D)
